# Initial kernel scaffold; baseline (speedup 1.0000x reference)
import jax
import jax.numpy as jnp
from jax.experimental import pallas as pl


def kernel(x, weight, bias):
    raise NotImplementedError("write your pallas kernel here")


# speedup vs baseline: 1.0615x; 1.0615x over previous
import jax
import jax.numpy as jnp
from jax.experimental import pallas as pl

TOPK = 8
NG = 64
DIM = 2048
BM = 512


def _router_kernel(x_ref, w_ref, b_ref, idx_ref, wt_ref):
    x = x_ref[...]
    w = w_ref[...]
    logits = jax.lax.dot_general(
        x, w, (((1,), (1,)), ((), ())), preferred_element_type=jnp.float32
    )
    logits = logits + b_ref[...]

    m = jnp.max(logits, axis=-1, keepdims=True)
    e = jnp.exp(logits - m)
    s = e / jnp.sum(e, axis=-1, keepdims=True)

    lane = jax.lax.broadcasted_iota(jnp.int32, s.shape, 1)
    vals = []
    idxs = []
    cur = s
    for _ in range(TOPK):
        mk = jnp.max(cur, axis=-1, keepdims=True)
        ak = jnp.min(jnp.where(cur == mk, lane, NG), axis=-1, keepdims=True)
        vals.append(mk)
        idxs.append(ak)
        cur = jnp.where(lane == ak, jnp.float32(-1.0), cur)

    v = jnp.concatenate(vals, axis=-1)
    i = jnp.concatenate(idxs, axis=-1)
    denom = jnp.sum(v, axis=-1, keepdims=True) + jnp.float32(1e-20)
    wt_ref[...] = v / denom
    idx_ref[...] = i


def kernel(x, weight, bias):
    bsz, seq_len, h = x.shape
    tokens = bsz * seq_len
    xs = x.reshape(tokens, h)
    b2 = bias.reshape(1, NG)

    grid = (tokens // BM,)
    topk_idx, topk_weight = pl.pallas_call(
        _router_kernel,
        grid=grid,
        in_specs=[
            pl.BlockSpec((BM, DIM), lambda i: (i, 0)),
            pl.BlockSpec((NG, DIM), lambda i: (0, 0)),
            pl.BlockSpec((1, NG), lambda i: (0, 0)),
        ],
        out_specs=[
            pl.BlockSpec((BM, TOPK), lambda i: (i, 0)),
            pl.BlockSpec((BM, TOPK), lambda i: (i, 0)),
        ],
        out_shape=[
            jax.ShapeDtypeStruct((tokens, TOPK), jnp.int32),
            jax.ShapeDtypeStruct((tokens, TOPK), jnp.float32),
        ],
    )(xs, weight, b2)

    aux_loss = jnp.asarray(0.0, dtype=jnp.float32)
    return (topk_idx, topk_weight, aux_loss)


# speedup vs baseline: 1.3085x; 1.2327x over previous
import jax
import jax.numpy as jnp
from jax.experimental import pallas as pl

TOPK = 8
NG = 64
DIM = 2048
BM = 512


def _router_kernel(x_ref, w_ref, b_ref, idx_ref, wt_ref):
    x = x_ref[...]
    w = w_ref[...]
    logits = jax.lax.dot_general(
        x, w, (((1,), (1,)), ((), ())), preferred_element_type=jnp.float32
    )
    logits = logits + b_ref[...]

    m = jnp.max(logits, axis=-1, keepdims=True)
    e = jnp.exp(logits - m)
    s = e / jnp.sum(e, axis=-1, keepdims=True)

    bits = jax.lax.bitcast_convert_type(s, jnp.int32)
    lane = jax.lax.broadcasted_iota(jnp.int32, (BM, NG), 1)
    key = (bits & jnp.int32(~63)) | (jnp.int32(63) - lane)

    kcols = []
    cur = key
    for _ in range(TOPK):
        km = jnp.max(cur, axis=-1, keepdims=True)
        kcols.append(km)
        cur = jnp.where(cur == km, jnp.int32(-1), cur)

    kcat = jnp.concatenate(kcols, axis=-1)
    ak = jnp.int32(63) - (kcat & jnp.int32(63))
    vb = kcat & jnp.int32(~63)
    v = jax.lax.bitcast_convert_type(vb, jnp.float32)

    denom = jnp.sum(v, axis=-1, keepdims=True) + jnp.float32(1e-20)
    wt_ref[...] = v / denom
    idx_ref[...] = ak


def kernel(x, weight, bias):
    bsz, seq_len, h = x.shape
    tokens = bsz * seq_len
    xs = x.reshape(tokens, h)
    b2 = bias.reshape(1, NG)

    grid = (tokens // BM,)
    topk_idx, topk_weight = pl.pallas_call(
        _router_kernel,
        grid=grid,
        in_specs=[
            pl.BlockSpec((BM, DIM), lambda i: (i, 0)),
            pl.BlockSpec((NG, DIM), lambda i: (0, 0)),
            pl.BlockSpec((1, NG), lambda i: (0, 0)),
        ],
        out_specs=[
            pl.BlockSpec((BM, TOPK), lambda i: (i, 0)),
            pl.BlockSpec((BM, TOPK), lambda i: (i, 0)),
        ],
        out_shape=[
            jax.ShapeDtypeStruct((tokens, TOPK), jnp.int32),
            jax.ShapeDtypeStruct((tokens, TOPK), jnp.float32),
        ],
    )(xs, weight, b2)

    aux_loss = jnp.asarray(0.0, dtype=jnp.float32)
    return (topk_idx, topk_weight, aux_loss)


# speedup vs baseline: 2.4764x; 1.8925x over previous
import jax
import jax.numpy as jnp
from jax.experimental import pallas as pl

TOPK = 8
NG = 64
DIM = 2048
BM = 1024


def _router_kernel(x_ref, w_ref, b_ref, idx_ref, wt_ref):
    x = x_ref[...]
    w = w_ref[...]
    lt = jax.lax.dot_general(
        w, x, (((1,), (1,)), ((), ())), preferred_element_type=jnp.float32
    )
    lt = lt + b_ref[:, :1]

    m = jnp.max(lt, axis=0, keepdims=True)
    e = jnp.exp(lt - m)
    s = e / jnp.sum(e, axis=0, keepdims=True)

    bits = jax.lax.bitcast_convert_type(s, jnp.int32)
    row = jax.lax.broadcasted_iota(jnp.int32, (NG, BM), 0)
    key = (bits & jnp.int32(~63)) | (jnp.int32(63) - row)

    kcols = []
    cur = key
    for _ in range(TOPK):
        km = jnp.max(cur, axis=0, keepdims=True)
        kcols.append(km)
        cur = jnp.where(cur == km, jnp.int32(-1), cur)

    kcat = jnp.concatenate(kcols, axis=0)
    ak = jnp.int32(63) - (kcat & jnp.int32(63))
    vb = kcat & jnp.int32(~63)
    v = jax.lax.bitcast_convert_type(vb, jnp.float32)

    denom = jnp.sum(v, axis=0, keepdims=True) + jnp.float32(1e-20)
    wt_ref[...] = v / denom
    idx_ref[...] = ak


def kernel(x, weight, bias):
    bsz, seq_len, h = x.shape
    tokens = bsz * seq_len
    xs = x.reshape(tokens, h)
    b2 = jnp.broadcast_to(bias.reshape(NG, 1), (NG, 128))

    grid = (tokens // BM,)
    idx8, wt8 = pl.pallas_call(
        _router_kernel,
        grid=grid,
        in_specs=[
            pl.BlockSpec((BM, DIM), lambda i: (i, 0)),
            pl.BlockSpec((NG, DIM), lambda i: (0, 0)),
            pl.BlockSpec((NG, 128), lambda i: (0, 0)),
        ],
        out_specs=[
            pl.BlockSpec((TOPK, BM), lambda i: (0, i)),
            pl.BlockSpec((TOPK, BM), lambda i: (0, i)),
        ],
        out_shape=[
            jax.ShapeDtypeStruct((TOPK, tokens), jnp.int32),
            jax.ShapeDtypeStruct((TOPK, tokens), jnp.float32),
        ],
    )(xs, weight, b2)

    aux_loss = jnp.asarray(0.0, dtype=jnp.float32)
    return (idx8.T, wt8.T, aux_loss)
